# baseline (speedup 1.0000x reference)
import functools

import jax
import jax.numpy as jnp
from jax.experimental import pallas as pl
from jax.experimental.pallas import tpu as pltpu

N = 8192
D = 64
INV_TAU = 2.0
LOG2E = 1.4426950408889634

_BM = 256
_BQ = 1024
_NBUF = 6


def _proj_rows(x, w, b, scale):
    y = jax.lax.dot_general(x, w, (((1,), (1,)), ((), ())),
                            preferred_element_type=jnp.float32) + b
    y = jnp.where(y > 0, y, jnp.exp(jnp.minimum(y, 0.0)) - 1.0)
    inv = jax.lax.rsqrt(jnp.sum(y * y, axis=1, keepdims=True)) * scale
    return (y * inv).astype(jnp.bfloat16)


def _outer_kernel(v1_ref, v2_ref, wt_ref, b_ref, p_hbm, n_hbm,
                  loss_ref, z1_scr, z2_scr, cnt_ref, psum_ref, nsum_ref):

    def body(p_ref, n_ref):
        c = cnt_ref[0]
        i = c // 8
        q = c % 8

        @pl.when(c == 0)
        def _prologue():
            psum_ref[0] = 0.0
            nsum_ref[0] = 0.0
            wt = wt_ref[...]
            b = b_ref[...]
            z1_scr[...] = _proj_rows(v1_ref[...], wt, b, INV_TAU * LOG2E)
            z2_scr[...] = _proj_rows(v2_ref[...], wt, b, 1.0)

        z1b = z1_scr[pl.ds(i * _BM, _BM), :]
        z2q = z2_scr[pl.ds(q * _BQ, _BQ), :]
        dot = jax.lax.dot_general(
            z1b, z2q,
            (((1,), (1,)), ((), ())),
            preferred_element_type=jnp.float32,
        )
        s = jnp.exp2(dot)
        psum_ref[0] += jnp.sum(s * p_ref[...])
        nsum_ref[0] += jnp.sum(s * n_ref[...])
        cnt_ref[0] = c + 1

        @pl.when(c == (N // _BM) * 8 - 1)
        def _epilogue():
            ps = psum_ref[0]
            loss_ref[0, 0] = jnp.log(ps + nsum_ref[0]) - jnp.log(ps)

    spec = lambda: pl.BlockSpec((_BM, _BQ), lambda i, q: (i, q),
                                pipeline_mode=pl.Buffered(buffer_count=_NBUF))
    pipe = pltpu.emit_pipeline(
        body,
        grid=(N // _BM, 8),
        in_specs=[spec(), spec()],
    )
    cnt_ref[0] = 0
    pipe(p_hbm, n_hbm)


@functools.partial(jax.jit, static_argnames=())
def kernel(v1_embs, v2_embs, pos, neg, W, b):
    b2 = b.reshape(1, D)

    loss = pl.pallas_call(
        _outer_kernel,
        in_specs=[
            pl.BlockSpec((N, D), lambda: (0, 0)),
            pl.BlockSpec((N, D), lambda: (0, 0)),
            pl.BlockSpec((D, D), lambda: (0, 0)),
            pl.BlockSpec((1, D), lambda: (0, 0)),
            pl.BlockSpec(memory_space=pl.ANY),
            pl.BlockSpec(memory_space=pl.ANY),
        ],
        out_specs=pl.BlockSpec(memory_space=pltpu.SMEM),
        out_shape=jax.ShapeDtypeStruct((1, 1), jnp.float32),
        scratch_shapes=[
            pltpu.VMEM((N, D), jnp.bfloat16),
            pltpu.VMEM((N, D), jnp.bfloat16),
            pltpu.SMEM((1,), jnp.int32),
            pltpu.SMEM((1,), jnp.float32),
            pltpu.SMEM((1,), jnp.float32),
        ],
        compiler_params=pltpu.CompilerParams(vmem_limit_bytes=63 * 1024 * 1024),
    )(v1_embs, v2_embs, W, b2, pos, neg)

    return loss[0, 0]


# speedup vs baseline: 1.1587x; 1.1587x over previous
import functools

import jax
import jax.numpy as jnp
from jax.experimental import pallas as pl
from jax.experimental.pallas import tpu as pltpu

N = 8192
D = 64
INV_TAU = 2.0
LOG2E = 1.4426950408889634

_BM = 256
_BQ = 2048
_NBUF = 6


def _proj_rows(x, w, b, scale):
    y = jax.lax.dot_general(x, w, (((1,), (1,)), ((), ())),
                            preferred_element_type=jnp.float32) + b
    y = jnp.where(y > 0, y, jnp.exp(jnp.minimum(y, 0.0)) - 1.0)
    inv = jax.lax.rsqrt(jnp.sum(y * y, axis=1, keepdims=True)) * scale
    return (y * inv).astype(jnp.bfloat16)


def _outer_kernel(v1_ref, v2_ref, wt_ref, b_ref, p_hbm, n_hbm,
                  loss_ref, z1_scr, z2_scr, cnt_ref, psum_ref, nsum_ref):

    def body(p_ref, n_ref):
        c = cnt_ref[0]
        i = c // 4
        q = c % 4

        @pl.when(c == 0)
        def _prologue():
            psum_ref[0] = 0.0
            nsum_ref[0] = 0.0
            wt = wt_ref[...]
            b = b_ref[...]
            z1_scr[...] = _proj_rows(v1_ref[...], wt, b, INV_TAU * LOG2E)
            z2_scr[...] = _proj_rows(v2_ref[...], wt, b, 1.0)

        z1b = z1_scr[pl.ds(i * _BM, _BM), :]
        z2q = z2_scr[pl.ds(q * _BQ, _BQ), :]
        dot = jax.lax.dot_general(
            z1b, z2q,
            (((1,), (1,)), ((), ())),
            preferred_element_type=jnp.float32,
        )
        s = jnp.exp2(dot)
        psum_ref[0] += jnp.sum(s * p_ref[...])
        nsum_ref[0] += jnp.sum(s * n_ref[...])
        cnt_ref[0] = c + 1

        @pl.when(c == (N // _BM) * 4 - 1)
        def _epilogue():
            ps = psum_ref[0]
            loss_ref[0, 0] = jnp.log(ps + nsum_ref[0]) - jnp.log(ps)

    spec = lambda: pl.BlockSpec((_BM, _BQ), lambda i, q: (i, q),
                                pipeline_mode=pl.Buffered(buffer_count=_NBUF))
    pipe = pltpu.emit_pipeline(
        body,
        grid=(N // _BM, 4),
        in_specs=[spec(), spec()],
    )
    cnt_ref[0] = 0
    pipe(p_hbm, n_hbm)


@functools.partial(jax.jit, static_argnames=())
def kernel(v1_embs, v2_embs, pos, neg, W, b):
    b2 = b.reshape(1, D)

    loss = pl.pallas_call(
        _outer_kernel,
        in_specs=[
            pl.BlockSpec((N, D), lambda: (0, 0)),
            pl.BlockSpec((N, D), lambda: (0, 0)),
            pl.BlockSpec((D, D), lambda: (0, 0)),
            pl.BlockSpec((1, D), lambda: (0, 0)),
            pl.BlockSpec(memory_space=pl.ANY),
            pl.BlockSpec(memory_space=pl.ANY),
        ],
        out_specs=pl.BlockSpec(memory_space=pltpu.SMEM),
        out_shape=jax.ShapeDtypeStruct((1, 1), jnp.float32),
        scratch_shapes=[
            pltpu.VMEM((N, D), jnp.bfloat16),
            pltpu.VMEM((N, D), jnp.bfloat16),
            pltpu.SMEM((1,), jnp.int32),
            pltpu.SMEM((1,), jnp.float32),
            pltpu.SMEM((1,), jnp.float32),
        ],
        compiler_params=pltpu.CompilerParams(vmem_limit_bytes=63 * 1024 * 1024),
    )(v1_embs, v2_embs, W, b2, pos, neg)

    return loss[0, 0]
